# initial kernel scaffold (unmeasured)
import jax
import jax.numpy as jnp
from jax import lax
from jax.experimental import pallas as pl
from jax.experimental.pallas import tpu as pltpu

N_DEV = 16

_CYCLE = [0, 1, 5, 9, 13, 14, 10, 6, 2, 3, 7, 11, 15, 12, 8, 4]


def kernel(x, w_mat, scale_x, scale_w):
    m_per, k = x.shape
    _, n_per = w_mat.shape

    perm = jnp.array(_CYCLE, dtype=jnp.int32)
    inv = jnp.zeros((N_DEV,), jnp.int32).at[perm].set(
        jnp.arange(N_DEV, dtype=jnp.int32)
    )
    my = lax.axis_index("i")
    r = inv[my]
    right = perm[(r + 1) % N_DEV]
    left = perm[(r - 1) % N_DEV]
    nbrs = jnp.stack([right, left]).astype(jnp.int32)
    hops = jnp.arange(N_DEV, dtype=jnp.int32)
    origins = (perm[(r - hops) % N_DEV] * m_per).astype(jnp.int32)

    def body(nbr_ref, org_ref, x_ref, w_ref, sx_ref, sw_ref, out_ref,
             buf_ref, send_sems, recv_sems):
        right_d = nbr_ref[0]
        left_d = nbr_ref[1]

        barrier_sem = pltpu.get_barrier_semaphore()
        for d in (right_d, left_d):
            pl.semaphore_signal(
                barrier_sem, inc=1,
                device_id=(d,), device_id_type=pl.DeviceIdType.MESH,
            )
        pl.semaphore_wait(barrier_sem, 2)

        buf_ref[0, :, :] = x_ref[:, :]
        scale = sx_ref[0] * sw_ref[0]

        def compute(slot):
            acc = lax.dot_general(
                buf_ref[slot, :, :], w_ref[:, :],
                dimension_numbers=(((1,), (0,)), ((), ())),
                preferred_element_type=jnp.float32,
            )
            y = acc * scale
            out_ref[pl.ds(org_ref[slot], m_per), :] = y * jax.nn.sigmoid(y)

        for h in range(N_DEV - 1):
            rdma = pltpu.make_async_remote_copy(
                src_ref=buf_ref.at[h],
                dst_ref=buf_ref.at[h + 1],
                send_sem=send_sems.at[h],
                recv_sem=recv_sems.at[h],
                device_id=(right_d,),
                device_id_type=pl.DeviceIdType.MESH,
            )
            rdma.start()
            compute(h)
            rdma.wait()
        compute(N_DEV - 1)

    out_shape = jax.ShapeDtypeStruct((N_DEV * m_per, n_per), jnp.float32)
    return pl.pallas_call(
        body,
        out_shape=out_shape,
        in_specs=[
            pl.BlockSpec(memory_space=pltpu.SMEM),
            pl.BlockSpec(memory_space=pltpu.SMEM),
            pl.BlockSpec(memory_space=pltpu.VMEM),
            pl.BlockSpec(memory_space=pltpu.VMEM),
            pl.BlockSpec(memory_space=pltpu.SMEM),
            pl.BlockSpec(memory_space=pltpu.SMEM),
        ],
        out_specs=pl.BlockSpec(memory_space=pltpu.VMEM),
        scratch_shapes=[
            pltpu.VMEM((N_DEV, m_per, k), x.dtype),
            pltpu.SemaphoreType.DMA((N_DEV - 1,)),
            pltpu.SemaphoreType.DMA((N_DEV - 1,)),
        ],
        compiler_params=pltpu.CompilerParams(collective_id=0),
    )(nbrs, origins, x, w_mat, scale_x, scale_w)


# baseline (device time: 213652 ns/iter reference)
import jax
import jax.numpy as jnp
from jax import lax
from jax.experimental import pallas as pl
from jax.experimental.pallas import tpu as pltpu

N_DEV = 16

_CYCLE = [0, 1, 5, 9, 13, 14, 10, 6, 2, 3, 7, 11, 15, 12, 8, 4]


def kernel(x, w_mat, scale_x, scale_w):
    m_per, k = x.shape
    _, n_per = w_mat.shape

    perm = jnp.array(_CYCLE, dtype=jnp.int32)
    inv = jnp.zeros((N_DEV,), jnp.int32).at[perm].set(
        jnp.arange(N_DEV, dtype=jnp.int32)
    )
    my = lax.axis_index("i")
    r = inv[my]
    right = perm[(r + 1) % N_DEV]
    left = perm[(r - 1) % N_DEV]
    nbrs = jnp.stack([right, left]).astype(jnp.int32)
    hops = jnp.arange(N_DEV, dtype=jnp.int32)
    origins = (perm[(r - hops) % N_DEV] * m_per).astype(jnp.int32)

    def body(nbr_ref, org_ref, x_ref, w_ref, sx_ref, sw_ref, out_ref,
             buf_ref, w_bf_ref, send_sems, recv_sems):
        right_d = nbr_ref[0]
        left_d = nbr_ref[1]

        barrier_sem = pltpu.get_barrier_semaphore()
        for d in (right_d, left_d):
            pl.semaphore_signal(
                barrier_sem, inc=1,
                device_id=(d,), device_id_type=pl.DeviceIdType.MESH,
            )
        pl.semaphore_wait(barrier_sem, 2)

        buf_ref[0, :, :] = x_ref[:, :].astype(jnp.float8_e4m3fn)
        w_bf_ref[:, :] = w_ref[:, :].astype(jnp.bfloat16)
        scale = sx_ref[0] * sw_ref[0]

        def compute(slot):
            acc = lax.dot_general(
                buf_ref[slot, :, :].astype(jnp.bfloat16), w_bf_ref[:, :],
                dimension_numbers=(((1,), (0,)), ((), ())),
                preferred_element_type=jnp.float32,
            )
            y = acc * scale
            off = pl.multiple_of(org_ref[slot], m_per)
            out_ref[pl.ds(off, m_per), :] = y * jax.nn.sigmoid(y)

        for h in range(N_DEV - 1):
            rdma = pltpu.make_async_remote_copy(
                src_ref=buf_ref.at[h],
                dst_ref=buf_ref.at[h + 1],
                send_sem=send_sems.at[h],
                recv_sem=recv_sems.at[h],
                device_id=(right_d,),
                device_id_type=pl.DeviceIdType.MESH,
            )
            rdma.start()
            compute(h)
            rdma.wait()
        compute(N_DEV - 1)

    out_shape = jax.ShapeDtypeStruct((N_DEV * m_per, n_per), jnp.float32)
    return pl.pallas_call(
        body,
        out_shape=out_shape,
        in_specs=[
            pl.BlockSpec(memory_space=pltpu.SMEM),
            pl.BlockSpec(memory_space=pltpu.SMEM),
            pl.BlockSpec(memory_space=pltpu.VMEM),
            pl.BlockSpec(memory_space=pltpu.VMEM),
            pl.BlockSpec(memory_space=pltpu.SMEM),
            pl.BlockSpec(memory_space=pltpu.SMEM),
        ],
        out_specs=pl.BlockSpec(memory_space=pltpu.VMEM),
        scratch_shapes=[
            pltpu.VMEM((N_DEV, m_per, k), jnp.float8_e4m3fn),
            pltpu.VMEM((k, n_per), jnp.bfloat16),
            pltpu.SemaphoreType.DMA((N_DEV - 1,)),
            pltpu.SemaphoreType.DMA((N_DEV - 1,)),
        ],
        compiler_params=pltpu.CompilerParams(collective_id=0),
    )(nbrs, origins, x, w_mat, scale_x, scale_w)


# device time: 124888 ns/iter; 1.7107x vs baseline; 1.7107x over previous
import jax
import jax.numpy as jnp
from jax import lax
from jax.experimental import pallas as pl
from jax.experimental.pallas import tpu as pltpu

N_DEV = 16

_CYCLE = [0, 1, 5, 9, 13, 14, 10, 6, 2, 3, 7, 11, 15, 12, 8, 4]


def kernel(x, w_mat, scale_x, scale_w):
    m_per, k = x.shape
    _, n_per = w_mat.shape

    perm = jnp.array(_CYCLE, dtype=jnp.int32)
    inv = jnp.zeros((N_DEV,), jnp.int32).at[perm].set(
        jnp.arange(N_DEV, dtype=jnp.int32)
    )
    my = lax.axis_index("i")
    r = inv[my]
    right = perm[(r + 1) % N_DEV]
    left = perm[(r - 1) % N_DEV]
    nbrs = jnp.stack([right, left]).astype(jnp.int32)
    hops = jnp.arange(N_DEV, dtype=jnp.int32)
    origins = (perm[(r - hops) % N_DEV] * m_per).astype(jnp.int32)

    def body(nbr_ref, org_ref, x_ref, w_ref, sx_ref, sw_ref, out_ref,
             buf_ref, w_bf_ref, fsend_sems, frecv_sems, bsend_sems,
             brecv_sems):
        right_d = nbr_ref[0]
        left_d = nbr_ref[1]

        barrier_sem = pltpu.get_barrier_semaphore()
        for d in (right_d, left_d):
            pl.semaphore_signal(
                barrier_sem, inc=1,
                device_id=(d,), device_id_type=pl.DeviceIdType.MESH,
            )
        pl.semaphore_wait(barrier_sem, 2)

        buf_ref[0, :, :] = x_ref[:, :].astype(jnp.float8_e4m3fn)
        w_bf_ref[:, :] = w_ref[:, :].astype(jnp.bfloat16)
        scale = sx_ref[0] * sw_ref[0]

        def compute(slot):
            acc = lax.dot_general(
                buf_ref[slot, :, :].astype(jnp.bfloat16), w_bf_ref[:, :],
                dimension_numbers=(((1,), (0,)), ((), ())),
                preferred_element_type=jnp.float32,
            )
            y = acc * scale
            off = pl.multiple_of(org_ref[slot], m_per)
            out_ref[pl.ds(off, m_per), :] = y * jax.nn.sigmoid(y)

        for h in range(1, 9):
            fwd = pltpu.make_async_remote_copy(
                src_ref=buf_ref.at[h - 1],
                dst_ref=buf_ref.at[h],
                send_sem=fsend_sems.at[h - 1],
                recv_sem=frecv_sems.at[h - 1],
                device_id=(right_d,),
                device_id_type=pl.DeviceIdType.MESH,
            )
            fwd.start()
            if h <= 7:
                bwd = pltpu.make_async_remote_copy(
                    src_ref=buf_ref.at[(17 - h) % N_DEV],
                    dst_ref=buf_ref.at[16 - h],
                    send_sem=bsend_sems.at[h - 1],
                    recv_sem=brecv_sems.at[h - 1],
                    device_id=(left_d,),
                    device_id_type=pl.DeviceIdType.MESH,
                )
                bwd.start()
            if h == 1:
                compute(0)
            else:
                compute(h - 1)
                compute(16 - (h - 1))
            fwd.wait()
            if h <= 7:
                bwd.wait()
        compute(8)

    out_shape = jax.ShapeDtypeStruct((N_DEV * m_per, n_per), jnp.float32)
    return pl.pallas_call(
        body,
        out_shape=out_shape,
        in_specs=[
            pl.BlockSpec(memory_space=pltpu.SMEM),
            pl.BlockSpec(memory_space=pltpu.SMEM),
            pl.BlockSpec(memory_space=pltpu.VMEM),
            pl.BlockSpec(memory_space=pltpu.VMEM),
            pl.BlockSpec(memory_space=pltpu.SMEM),
            pl.BlockSpec(memory_space=pltpu.SMEM),
        ],
        out_specs=pl.BlockSpec(memory_space=pltpu.VMEM),
        scratch_shapes=[
            pltpu.VMEM((N_DEV, m_per, k), jnp.float8_e4m3fn),
            pltpu.VMEM((k, n_per), jnp.bfloat16),
            pltpu.SemaphoreType.DMA((8,)),
            pltpu.SemaphoreType.DMA((8,)),
            pltpu.SemaphoreType.DMA((7,)),
            pltpu.SemaphoreType.DMA((7,)),
        ],
        compiler_params=pltpu.CompilerParams(collective_id=0),
    )(nbrs, origins, x, w_mat, scale_x, scale_w)


# device time: 108527 ns/iter; 1.9687x vs baseline; 1.1508x over previous
import jax
import jax.numpy as jnp
from jax import lax
from jax.experimental import pallas as pl
from jax.experimental.pallas import tpu as pltpu

N_DEV = 16

_CYCLE = [0, 1, 5, 9, 13, 14, 10, 6, 2, 3, 7, 11, 15, 12, 8, 4]


def kernel(x, w_mat, scale_x, scale_w):
    m_per, k = x.shape
    _, n_per = w_mat.shape

    perm = jnp.array(_CYCLE, dtype=jnp.int32)
    inv = jnp.zeros((N_DEV,), jnp.int32).at[perm].set(
        jnp.arange(N_DEV, dtype=jnp.int32)
    )
    my = lax.axis_index("i")
    r = inv[my]
    right = perm[(r + 1) % N_DEV]
    left = perm[(r - 1) % N_DEV]
    nbrs = jnp.stack([right, left]).astype(jnp.int32)
    hops = jnp.arange(N_DEV, dtype=jnp.int32)
    origins = (perm[(r - hops) % N_DEV] * m_per).astype(jnp.int32)

    def body(nbr_ref, org_ref, x_ref, w_ref, sx_ref, sw_ref, out_ref,
             buf_ref, w_bf_ref, fsend_sems, frecv_sems, bsend_sems,
             brecv_sems):
        right_d = nbr_ref[0]
        left_d = nbr_ref[1]

        barrier_sem = pltpu.get_barrier_semaphore()
        for d in (right_d, left_d):
            pl.semaphore_signal(
                barrier_sem, inc=1,
                device_id=(d,), device_id_type=pl.DeviceIdType.MESH,
            )
        pl.semaphore_wait(barrier_sem, 2)

        buf_ref[0, :, :] = x_ref[:, :].astype(jnp.float8_e4m3fn)
        w_bf_ref[:, :] = w_ref[:, :].astype(jnp.bfloat16)
        scale = sx_ref[0] * sw_ref[0]

        def compute(slot):
            acc = lax.dot_general(
                buf_ref[slot, :, :].astype(jnp.bfloat16), w_bf_ref[:, :],
                dimension_numbers=(((1,), (0,)), ((), ())),
                preferred_element_type=jnp.float32,
            )
            y = acc * scale
            off = pl.multiple_of(org_ref[slot], m_per)
            out_ref[pl.ds(off, m_per), :] = y * jax.nn.sigmoid(y)

        half = m_per // 2

        def make(direction, h, j):
            if direction == "f":
                src, dst, dev = h - 1, h, right_d
                ssem, rsem = fsend_sems, frecv_sems
            else:
                src, dst, dev = (17 - h) % N_DEV, 16 - h, left_d
                ssem, rsem = bsend_sems, brecv_sems
            i = (h - 1) * 2 + j
            return pltpu.make_async_remote_copy(
                src_ref=buf_ref.at[src, pl.ds(j * half, half), :],
                dst_ref=buf_ref.at[dst, pl.ds(j * half, half), :],
                send_sem=ssem.at[i],
                recv_sem=rsem.at[i],
                device_id=(dev,),
                device_id_type=pl.DeviceIdType.MESH,
            )

        def msgs(h):
            out = []
            for j in (0, 1):
                if not (h == 8 and j == 1):
                    out.append(("f", h, j))
                if not (h == 8 and j == 0):
                    out.append(("b", h, j))
            return out

        sent = []
        prev = {}
        for d, h, j in msgs(1):
            rdma = make(d, h, j)
            rdma.start()
            sent.append(rdma)
            prev[(d, j)] = rdma
        compute(0)
        for h in range(2, 9):
            this_hop = msgs(h)
            for d, j in (("f", 0), ("b", 0), ("f", 1), ("b", 1)):
                prev[(d, j)].wait_recv()
                if (d, h, j) in this_hop:
                    rdma = make(d, h, j)
                    rdma.start()
                    sent.append(rdma)
                    prev[(d, j)] = rdma
            compute(h - 1)
            compute(17 - h)
        prev[("f", 0)].wait_recv()
        prev[("b", 1)].wait_recv()
        compute(8)
        for rdma in sent:
            rdma.wait_send()

    out_shape = jax.ShapeDtypeStruct((N_DEV * m_per, n_per), jnp.float32)
    return pl.pallas_call(
        body,
        out_shape=out_shape,
        in_specs=[
            pl.BlockSpec(memory_space=pltpu.SMEM),
            pl.BlockSpec(memory_space=pltpu.SMEM),
            pl.BlockSpec(memory_space=pltpu.VMEM),
            pl.BlockSpec(memory_space=pltpu.VMEM),
            pl.BlockSpec(memory_space=pltpu.SMEM),
            pl.BlockSpec(memory_space=pltpu.SMEM),
        ],
        out_specs=pl.BlockSpec(memory_space=pltpu.VMEM),
        scratch_shapes=[
            pltpu.VMEM((N_DEV, m_per, k), jnp.float8_e4m3fn),
            pltpu.VMEM((k, n_per), jnp.bfloat16),
            pltpu.SemaphoreType.DMA((16,)),
            pltpu.SemaphoreType.DMA((16,)),
            pltpu.SemaphoreType.DMA((16,)),
            pltpu.SemaphoreType.DMA((16,)),
        ],
        compiler_params=pltpu.CompilerParams(collective_id=0),
    )(nbrs, origins, x, w_mat, scale_x, scale_w)


# device time: 101423 ns/iter; 2.1065x vs baseline; 1.0700x over previous
import jax
import jax.numpy as jnp
from jax import lax
from jax.experimental import pallas as pl
from jax.experimental.pallas import tpu as pltpu

N_DEV = 16


def kernel(x, w_mat, scale_x, scale_w):
    m_per, k = x.shape
    _, n_per = w_mat.shape

    def body(x_ref, w_ref, sx_ref, sw_ref, out_ref,
             buf_ref, w_bf_ref, fsend_sems, frecv_sems, bsend_sems,
             brecv_sems):
        my = lax.axis_index("i")
        z = my // 4
        p = my % 4
        r = jnp.where(p == 0, (16 - z) % 16,
            jnp.where(p == 1, 1 + z,
            jnp.where(p == 2, 8 - z, 9 + z))).astype(jnp.int32)

        def log_of(q):
            return jnp.where(q == 0, 0,
                   jnp.where(q <= 4, 4 * (q - 1) + 1,
                   jnp.where(q <= 8, 4 * (8 - q) + 2,
                   jnp.where(q <= 12, 4 * (q - 9) + 3,
                             4 * (16 - q))))).astype(jnp.int32)

        right_d = log_of((r + 1) % N_DEV)
        left_d = log_of((r + N_DEV - 1) % N_DEV)
        org = [log_of((r - s + N_DEV) % N_DEV) * m_per for s in range(N_DEV)]

        barrier_sem = pltpu.get_barrier_semaphore()
        for d in (right_d, left_d):
            pl.semaphore_signal(
                barrier_sem, inc=1,
                device_id=(d,), device_id_type=pl.DeviceIdType.MESH,
            )
        pl.semaphore_wait(barrier_sem, 2)

        buf_ref[0, :, :] = x_ref[:, :].astype(jnp.float8_e4m3fn)
        scale = sx_ref[0] * sw_ref[0]

        def compute_rows(slot, row0, nrows):
            acc = lax.dot_general(
                buf_ref[slot, pl.ds(row0, nrows), :].astype(jnp.bfloat16),
                w_bf_ref[:, :],
                dimension_numbers=(((1,), (0,)), ((), ())),
                preferred_element_type=jnp.float32,
            )
            y = acc * scale
            off = pl.multiple_of(org[slot] + row0, nrows)
            out_ref[pl.ds(off, nrows), :] = y * jax.nn.sigmoid(y)

        def compute(slot):
            compute_rows(slot, 0, m_per)

        half = m_per // 2

        def make(direction, h, j):
            if direction == "f":
                src, dst, dev = h - 1, h, right_d
                ssem, rsem = fsend_sems, frecv_sems
            else:
                src, dst, dev = (17 - h) % N_DEV, 16 - h, left_d
                ssem, rsem = bsend_sems, brecv_sems
            i = (h - 1) * 2 + j
            return pltpu.make_async_remote_copy(
                src_ref=buf_ref.at[src, pl.ds(j * half, half), :],
                dst_ref=buf_ref.at[dst, pl.ds(j * half, half), :],
                send_sem=ssem.at[i],
                recv_sem=rsem.at[i],
                device_id=(dev,),
                device_id_type=pl.DeviceIdType.MESH,
            )

        def msgs(h):
            out = []
            for j in (0, 1):
                if not (h == 8 and j == 1):
                    out.append(("f", h, j))
                if not (h == 8 and j == 0):
                    out.append(("b", h, j))
            return out

        sent = []
        prev = {}
        for d, h, j in msgs(1):
            rdma = make(d, h, j)
            rdma.start()
            sent.append(rdma)
            prev[(d, j)] = rdma
        w_bf_ref[:, :] = w_ref[:, :].astype(jnp.bfloat16)
        compute(0)
        for h in range(2, 9):
            this_hop = msgs(h)
            for d, j in (("f", 0), ("b", 0), ("f", 1), ("b", 1)):
                prev[(d, j)].wait_recv()
                if (d, h, j) in this_hop:
                    rdma = make(d, h, j)
                    rdma.start()
                    sent.append(rdma)
                    prev[(d, j)] = rdma
            compute(h - 1)
            compute(17 - h)
        prev[("f", 0)].wait_recv()
        compute_rows(8, 0, half)
        prev[("b", 1)].wait_recv()
        compute_rows(8, half, half)
        for rdma in sent:
            rdma.wait_send()

    out_shape = jax.ShapeDtypeStruct((N_DEV * m_per, n_per), jnp.float32)
    return pl.pallas_call(
        body,
        out_shape=out_shape,
        in_specs=[
            pl.BlockSpec(memory_space=pltpu.VMEM),
            pl.BlockSpec(memory_space=pltpu.VMEM),
            pl.BlockSpec(memory_space=pltpu.SMEM),
            pl.BlockSpec(memory_space=pltpu.SMEM),
        ],
        out_specs=pl.BlockSpec(memory_space=pltpu.VMEM),
        scratch_shapes=[
            pltpu.VMEM((N_DEV, m_per, k), jnp.float8_e4m3fn),
            pltpu.VMEM((k, n_per), jnp.bfloat16),
            pltpu.SemaphoreType.DMA((16,)),
            pltpu.SemaphoreType.DMA((16,)),
            pltpu.SemaphoreType.DMA((16,)),
            pltpu.SemaphoreType.DMA((16,)),
        ],
        compiler_params=pltpu.CompilerParams(collective_id=0),
    )(x, w_mat, scale_x, scale_w)


# device time: 101286 ns/iter; 2.1094x vs baseline; 1.0014x over previous
import jax
import jax.numpy as jnp
from jax import lax
from jax.experimental import pallas as pl
from jax.experimental.pallas import tpu as pltpu

N_DEV = 16


def kernel(x, w_mat, scale_x, scale_w):
    m_per, k = x.shape
    _, n_per = w_mat.shape

    def body(x_ref, w_ref, sx_ref, sw_ref, out_ref,
             buf_ref, w_bf_ref, x_vmem, w_vmem, acc_ref,
             fsend_sems, frecv_sems, bsend_sems, brecv_sems,
             ld_sems, st_sems):
        my = lax.axis_index("i")
        z = my // 4
        p = my % 4
        r = jnp.where(p == 0, (16 - z) % 16,
            jnp.where(p == 1, 1 + z,
            jnp.where(p == 2, 8 - z, 9 + z))).astype(jnp.int32)

        def log_of(q):
            return jnp.where(q == 0, 0,
                   jnp.where(q <= 4, 4 * (q - 1) + 1,
                   jnp.where(q <= 8, 4 * (8 - q) + 2,
                   jnp.where(q <= 12, 4 * (q - 9) + 3,
                             4 * (16 - q))))).astype(jnp.int32)

        right_d = log_of((r + 1) % N_DEV)
        left_d = log_of((r + N_DEV - 1) % N_DEV)
        org = [log_of((r - s + N_DEV) % N_DEV) * m_per for s in range(N_DEV)]

        x_load = pltpu.make_async_copy(x_ref, x_vmem, ld_sems.at[0])
        x_load.start()
        w_load = pltpu.make_async_copy(w_ref, w_vmem, ld_sems.at[1])
        w_load.start()

        barrier_sem = pltpu.get_barrier_semaphore()
        for d in (right_d, left_d):
            pl.semaphore_signal(
                barrier_sem, inc=1,
                device_id=(d,), device_id_type=pl.DeviceIdType.MESH,
            )
        pl.semaphore_wait(barrier_sem, 2)

        x_load.wait()
        buf_ref[0, :, :] = x_vmem[:, :].astype(jnp.float8_e4m3fn)
        scale = sx_ref[0] * sw_ref[0]

        store_cps = []

        def compute_rows(slot, row0, nrows, st_idx):
            acc = lax.dot_general(
                buf_ref[slot, pl.ds(row0, nrows), :].astype(jnp.bfloat16),
                w_bf_ref[:, :],
                dimension_numbers=(((1,), (0,)), ((), ())),
                preferred_element_type=jnp.float32,
            )
            y = acc * scale
            off = pl.multiple_of(org[slot] + row0, nrows)
            acc_ref[pl.ds(off, nrows), :] = y * jax.nn.sigmoid(y)
            cp = pltpu.make_async_copy(
                acc_ref.at[pl.ds(off, nrows), :],
                out_ref.at[pl.ds(off, nrows), :],
                st_sems.at[st_idx],
            )
            cp.start()
            store_cps.append(cp)

        def compute(slot):
            compute_rows(slot, 0, m_per, slot)

        half = m_per // 2

        def make(direction, h, j):
            if direction == "f":
                src, dst, dev = h - 1, h, right_d
                ssem, rsem = fsend_sems, frecv_sems
            else:
                src, dst, dev = (17 - h) % N_DEV, 16 - h, left_d
                ssem, rsem = bsend_sems, brecv_sems
            i = (h - 1) * 2 + j
            return pltpu.make_async_remote_copy(
                src_ref=buf_ref.at[src, pl.ds(j * half, half), :],
                dst_ref=buf_ref.at[dst, pl.ds(j * half, half), :],
                send_sem=ssem.at[i],
                recv_sem=rsem.at[i],
                device_id=(dev,),
                device_id_type=pl.DeviceIdType.MESH,
            )

        def msgs(h):
            out = []
            for j in (0, 1):
                if not (h == 8 and j == 1):
                    out.append(("f", h, j))
                if not (h == 8 and j == 0):
                    out.append(("b", h, j))
            return out

        sent = []
        prev = {}
        for d, h, j in msgs(1):
            rdma = make(d, h, j)
            rdma.start()
            sent.append(rdma)
            prev[(d, j)] = rdma
        w_load.wait()
        w_bf_ref[:, :] = w_vmem[:, :].astype(jnp.bfloat16)
        compute(0)
        for h in range(2, 9):
            this_hop = msgs(h)
            for d, j in (("f", 0), ("b", 0), ("f", 1), ("b", 1)):
                prev[(d, j)].wait_recv()
                if (d, h, j) in this_hop:
                    rdma = make(d, h, j)
                    rdma.start()
                    sent.append(rdma)
                    prev[(d, j)] = rdma
            compute(h - 1)
            compute(17 - h)
        prev[("f", 0)].wait_recv()
        compute_rows(8, 0, half, 8)
        prev[("b", 1)].wait_recv()
        compute_rows(8, half, half, 16)
        for rdma in sent:
            rdma.wait_send()
        for cp in store_cps:
            cp.wait()

    out_shape = jax.ShapeDtypeStruct((N_DEV * m_per, n_per), jnp.float32)
    return pl.pallas_call(
        body,
        out_shape=out_shape,
        in_specs=[
            pl.BlockSpec(memory_space=pl.ANY),
            pl.BlockSpec(memory_space=pl.ANY),
            pl.BlockSpec(memory_space=pltpu.SMEM),
            pl.BlockSpec(memory_space=pltpu.SMEM),
        ],
        out_specs=pl.BlockSpec(memory_space=pl.ANY),
        scratch_shapes=[
            pltpu.VMEM((N_DEV, m_per, k), jnp.float8_e4m3fn),
            pltpu.VMEM((k, n_per), jnp.bfloat16),
            pltpu.VMEM((m_per, k), jnp.float32),
            pltpu.VMEM((k, n_per), jnp.float32),
            pltpu.VMEM((N_DEV * m_per, n_per), jnp.float32),
            pltpu.SemaphoreType.DMA((16,)),
            pltpu.SemaphoreType.DMA((16,)),
            pltpu.SemaphoreType.DMA((16,)),
            pltpu.SemaphoreType.DMA((16,)),
            pltpu.SemaphoreType.DMA((2,)),
            pltpu.SemaphoreType.DMA((17,)),
        ],
        compiler_params=pltpu.CompilerParams(
            collective_id=0, vmem_limit_bytes=60 * 1024 * 1024,
        ),
    )(x, w_mat, scale_x, scale_w)
